# baseline (device time: 15983 ns/iter reference)
import jax
import jax.numpy as jnp
from jax import lax
from jax.experimental import pallas as pl
from jax.experimental.pallas import tpu as pltpu

N_DEV = 4
N_HALF = 2


def kernel(A, B):
    m_per, k = A.shape
    _, n = B.shape
    m_half = m_per // N_HALF

    def body(a_ref, b_ref, out_ref, a_bf_ref, b_bf_ref, comm_ref,
             send_sems, recv_sems):
        my = lax.axis_index("i")

        barrier_sem = pltpu.get_barrier_semaphore()
        for d in range(1, N_DEV):
            peer = lax.rem(my + d, N_DEV)
            pl.semaphore_signal(
                barrier_sem, inc=1,
                device_id=(peer,), device_id_type=pl.DeviceIdType.MESH,
            )

        a_bf_ref[...] = a_ref[...].reshape(N_HALF, m_half, k).astype(
            jnp.bfloat16
        )
        b_bf_ref[...] = b_ref[...].astype(jnp.bfloat16)

        pl.semaphore_wait(barrier_sem, N_DEV - 1)

        rdmas = {}
        for d in (2, 1, 3):
            peer = lax.rem(my + d, N_DEV)
            for j in range(N_HALF):
                s = (d - 1) * N_HALF + j
                rdma = pltpu.make_async_remote_copy(
                    src_ref=a_bf_ref.at[j],
                    dst_ref=comm_ref.at[s],
                    send_sem=send_sems.at[s],
                    recv_sem=recv_sems.at[s],
                    device_id=(peer,),
                    device_id_type=pl.DeviceIdType.MESH,
                )
                rdma.start()
                rdmas[s] = rdma

        for j in range(N_HALF):
            out_ref[pl.ds(my * m_per + j * m_half, m_half), :] = jnp.dot(
                a_bf_ref[j], b_bf_ref[...],
                preferred_element_type=jnp.float32,
            )

        for d in (1, 3, 2):
            origin = lax.rem(my + N_DEV - d, N_DEV)
            for j in range(N_HALF):
                s = (d - 1) * N_HALF + j
                rdmas[s].wait_recv()
                out_ref[pl.ds(origin * m_per + j * m_half, m_half), :] = (
                    jnp.dot(
                        comm_ref[s], b_bf_ref[...],
                        preferred_element_type=jnp.float32,
                    )
                )

        for s in range(N_HALF * (N_DEV - 1)):
            rdmas[s].wait_send()

    return pl.pallas_call(
        body,
        out_shape=jax.ShapeDtypeStruct((N_DEV * m_per, n), jnp.float32),
        in_specs=[
            pl.BlockSpec(memory_space=pltpu.VMEM),
            pl.BlockSpec(memory_space=pltpu.VMEM),
        ],
        out_specs=pl.BlockSpec(memory_space=pltpu.VMEM),
        scratch_shapes=[
            pltpu.VMEM((N_HALF, m_half, k), jnp.bfloat16),
            pltpu.VMEM((k, n), jnp.bfloat16),
            pltpu.VMEM((N_HALF * (N_DEV - 1), m_half, k), jnp.bfloat16),
            pltpu.SemaphoreType.DMA((N_HALF * (N_DEV - 1),)),
            pltpu.SemaphoreType.DMA((N_HALF * (N_DEV - 1),)),
        ],
        compiler_params=pltpu.CompilerParams(collective_id=0),
    )(A, B)


# device time: 4971 ns/iter; 3.2152x vs baseline; 3.2152x over previous
import jax
import jax.numpy as jnp
from jax import lax
from jax.experimental import pallas as pl
from jax.experimental.pallas import tpu as pltpu

N_DEV = 4


def kernel(A, B):
    m_per, k = A.shape
    _, n = B.shape

    def body(a_ref, b_ref, out_ref, a_bf_ref, b_bf_ref, comm_ref,
             send_sems, recv_sems):
        my = lax.axis_index("i")

        barrier_sem = pltpu.get_barrier_semaphore()
        for d in range(1, N_DEV):
            peer = lax.rem(my + d, N_DEV)
            pl.semaphore_signal(
                barrier_sem, inc=1,
                device_id=(peer,), device_id_type=pl.DeviceIdType.MESH,
            )

        a_bf_ref[...] = a_ref[...].astype(jnp.bfloat16)
        b_bf_ref[...] = b_ref[...].astype(jnp.bfloat16)

        pl.semaphore_wait(barrier_sem, N_DEV - 1)

        rdmas = {}
        for d in (2, 1, 3):
            peer = lax.rem(my + d, N_DEV)
            rdma = pltpu.make_async_remote_copy(
                src_ref=a_bf_ref,
                dst_ref=comm_ref.at[d - 1],
                send_sem=send_sems.at[d - 1],
                recv_sem=recv_sems.at[d - 1],
                device_id=(peer,),
                device_id_type=pl.DeviceIdType.MESH,
            )
            rdma.start()
            rdmas[d] = rdma

        out_ref[pl.ds(my * m_per, m_per), :] = jnp.dot(
            a_bf_ref[...], b_bf_ref[...], preferred_element_type=jnp.float32
        )

        for d in (1, 3, 2):
            rdmas[d].wait_recv()
            origin = lax.rem(my + N_DEV - d, N_DEV)
            out_ref[pl.ds(origin * m_per, m_per), :] = jnp.dot(
                comm_ref[d - 1], b_bf_ref[...],
                preferred_element_type=jnp.float32,
            )

        for d in range(1, N_DEV):
            rdmas[d].wait_send()

    return pl.pallas_call(
        body,
        out_shape=jax.ShapeDtypeStruct((N_DEV * m_per, n), jnp.float32),
        in_specs=[
            pl.BlockSpec(memory_space=pltpu.VMEM),
            pl.BlockSpec(memory_space=pltpu.VMEM),
        ],
        out_specs=pl.BlockSpec(memory_space=pltpu.VMEM),
        scratch_shapes=[
            pltpu.VMEM((m_per, k), jnp.bfloat16),
            pltpu.VMEM((k, n), jnp.bfloat16),
            pltpu.VMEM((N_DEV - 1, m_per, k), jnp.bfloat16),
            pltpu.SemaphoreType.DMA((N_DEV - 1,)),
            pltpu.SemaphoreType.DMA((N_DEV - 1,)),
        ],
        compiler_params=pltpu.CompilerParams(collective_id=0),
    )(A, B)
